# baseline (speedup 1.0000x reference)
import functools

import jax
import jax.numpy as jnp
from jax import lax
from jax.experimental import pallas as pl
from jax.experimental.pallas import tpu as pltpu
from jax.experimental.pallas import tpu_sc as plsc

_BATCH = 16384
_EMB = 32


def _make_sc_gather_t(batch, emb_dim, vocab):
    info = plsc.get_sparse_core_info()
    nc, ns = info.num_cores, info.num_subcores
    half = emb_dim // nc
    ichunk = 2048
    nt = batch // 128
    mesh = plsc.VectorSubcoreMesh(core_axis_name="c", subcore_axis_name="s")

    @functools.partial(
        pl.kernel,
        out_type=jax.ShapeDtypeStruct((2 * emb_dim, nt, 128), jnp.float32),
        mesh=mesh,
        compiler_params=pltpu.CompilerParams(needs_layout_passes=False),
        scratch_types=[
            pltpu.VMEM((vocab,), jnp.float32),
            pltpu.VMEM((2, ichunk), jnp.int32),
            pltpu.VMEM((2, ichunk), jnp.int32),
            pltpu.VMEM((2, ichunk // 128, 128), jnp.float32),
            pltpu.VMEM((2, ichunk // 128, 128), jnp.float32),
            pltpu.SemaphoreType.DMA,
            pltpu.SemaphoreType.DMA,
            pltpu.SemaphoreType.DMA,
            pltpu.SemaphoreType.DMA,
            pltpu.SemaphoreType.DMA,
            pltpu.SemaphoreType.DMA,
            pltpu.SemaphoreType.DMA,
            pltpu.SemaphoreType.DMA,
            pltpu.SemaphoreType.DMA,
        ],
    )
    def sc_gather(tT_hbm, ia_hbm, ib_hbm, rows_hbm,
                  row_v, ia_v, ib_v, ra_v, rb_v,
                  sem_row, sem_ia0, sem_ia1, sem_ib0, sem_ib1,
                  sem_ra0, sem_ra1, sem_rb0, sem_rb1):
        c = lax.axis_index("c")
        s = lax.axis_index("s")
        d = c * half + s
        sem_ia = [sem_ia0, sem_ia1]
        sem_ib = [sem_ib0, sem_ib1]
        sem_ra = [sem_ra0, sem_ra1]
        sem_rb = [sem_rb0, sem_rb1]

        h_row = pltpu.async_copy(tT_hbm.at[d], row_v, sem_row)

        nch = batch // ichunk
        tpc = ichunk // 128

        def start_in(k):
            sl = k % 2
            return (
                pltpu.async_copy(ia_hbm.at[pl.ds(k * ichunk, ichunk)],
                                 ia_v.at[sl], sem_ia[sl]),
                pltpu.async_copy(ib_hbm.at[pl.ds(k * ichunk, ichunk)],
                                 ib_v.at[sl], sem_ib[sl]),
            )

        h_in = start_in(0)
        h_row.wait()
        h_out = [None] * nch
        for k in range(nch):
            sl = k % 2
            h_in[0].wait()
            h_in[1].wait()
            if k + 1 < nch:
                h_in = start_in(k + 1)
            if k >= 2:
                h_out[k - 2][0].wait()
                h_out[k - 2][1].wait()

            @plsc.parallel_loop(0, ichunk // 16, unroll=8)
            def gather_one(j):
                off = j * 16
                t, l = off // 128, off % 128
                ra_v[sl, t, pl.ds(l, 16)] = plsc.load_gather(
                    row_v, [ia_v[sl, pl.ds(off, 16)]])
                rb_v[sl, t, pl.ds(l, 16)] = plsc.load_gather(
                    row_v, [ib_v[sl, pl.ds(off, 16)]])

            h_out[k] = (
                pltpu.async_copy(ra_v.at[sl],
                                 rows_hbm.at[d, pl.ds(k * tpc, tpc)],
                                 sem_ra[sl]),
                pltpu.async_copy(rb_v.at[sl],
                                 rows_hbm.at[emb_dim + d, pl.ds(k * tpc, tpc)],
                                 sem_rb[sl]),
            )
        for k in (nch - 2, nch - 1):
            h_out[k][0].wait()
            h_out[k][1].wait()

    return sc_gather


def _make_sc_xpose(batch, emb_dim):
    info = plsc.get_sparse_core_info()
    nw = info.num_cores * info.num_subcores
    nt = batch // 128
    per_w = batch // nw
    tpw = per_w // 128
    two_d = 2 * emb_dim
    mesh = plsc.VectorSubcoreMesh(core_axis_name="c", subcore_axis_name="s")

    @functools.partial(
        pl.kernel,
        out_type=jax.ShapeDtypeStruct((batch, 128), jnp.float32),
        mesh=mesh,
        compiler_params=pltpu.CompilerParams(use_tc_tiling_on_sc=False,
                                             needs_layout_passes=False),
        scratch_types=[
            pltpu.VMEM((two_d, tpw, 128), jnp.float32),
            pltpu.VMEM((per_w, 128), jnp.float32),
        ],
    )
    def sc_xpose(rows_hbm, comb_hbm, xch_v, out_t):
        wid = lax.axis_index("s") * info.num_cores + lax.axis_index("c")
        b0 = wid * per_w
        pltpu.sync_copy(rows_hbm.at[:, pl.ds(wid * tpw, tpw)], xch_v)
        lane16 = lax.broadcasted_iota(jnp.int32, (16,), 0)

        @plsc.parallel_loop(0, two_d * per_w // 16, unroll=8)
        def xpose_one(t):
            r = t % two_d
            bb = (t // two_d) * 16
            vals = xch_v[r, bb // 128, pl.ds(bb % 128, 16)]
            plsc.store_scatter(
                out_t, [bb + lane16, jnp.full((16,), r, jnp.int32)], vals)
        pltpu.sync_copy(out_t, comb_hbm.at[pl.ds(b0, per_w)])

    return sc_xpose


_sc_gather_t = _make_sc_gather_t(_BATCH, _EMB, 100000)
_sc_xpose = _make_sc_xpose(_BATCH, _EMB)


def _dot_t(a, w):
    return lax.dot_general(a, w, (((1,), (1,)), ((), ())),
                           preferred_element_type=jnp.float32)


def _mlp_body(comb, ft, w1, b1, w2, b2, w3, b3, w4, out):
    x = jnp.concatenate([comb[:, 0:64], ft[...]], axis=1)
    h = jnp.maximum(_dot_t(x, w1[...]) + b1[...], 0.0)
    h = jnp.maximum(_dot_t(h, w2[...]) + b2[...], 0.0)
    h = jnp.maximum(_dot_t(h, w3[...]) + b3[...], 0.0)
    h = jnp.concatenate([h, jnp.ones((h.shape[0], 1), jnp.float32)], axis=1)
    out[...] = jax.nn.sigmoid(_dot_t(w4[...], h))


def _mlp(comb, ft, W1, b1, W2, b2, W3, b3, W4e, blk=2048):
    batch = comb.shape[0]
    grid = (batch // blk,)
    full = lambda a: pl.BlockSpec(a.shape, lambda i: (0,) * a.ndim)
    row = lambda a: pl.BlockSpec((blk, a.shape[1]), lambda i: (i, 0))
    return pl.pallas_call(
        _mlp_body,
        grid=grid,
        in_specs=[
            row(comb), row(ft),
            full(W1), full(b1), full(W2), full(b2),
            full(W3), full(b3), full(W4e),
        ],
        out_specs=pl.BlockSpec((1, blk), lambda i: (0, i)),
        out_shape=jax.ShapeDtypeStruct((1, batch), jnp.float32),
    )(comb, ft, W1, b1, W2, b2, W3, b3, W4e)


def kernel(airport_a, airport_b, features, table,
           W1, b1, W2, b2, W3, b3, W4, b4):
    ia = airport_a.astype(jnp.int32)
    ib = airport_b.astype(jnp.int32)
    rows = _sc_gather_t(table.T, ia, ib)
    comb = _sc_xpose(rows)
    w4e = jnp.concatenate([W4, b4.reshape(1, 1)], axis=1)
    out = _mlp(comb, features,
               W1, b1.reshape(1, -1), W2, b2.reshape(1, -1),
               W3, b3.reshape(1, -1), w4e)
    return out.reshape(-1)


# speedup vs baseline: 1.1581x; 1.1581x over previous
import functools

import jax
import jax.numpy as jnp
from jax import lax
from jax.experimental import pallas as pl
from jax.experimental.pallas import tpu as pltpu
from jax.experimental.pallas import tpu_sc as plsc

_BATCH = 16384
_EMB = 32


def _make_sc_gather_t(batch, emb_dim, vocab):
    info = plsc.get_sparse_core_info()
    nc, ns = info.num_cores, info.num_subcores
    half = emb_dim // nc
    ichunk = 2048
    nt = batch // 128
    mesh = plsc.VectorSubcoreMesh(core_axis_name="c", subcore_axis_name="s")

    @functools.partial(
        pl.kernel,
        out_type=jax.ShapeDtypeStruct((2 * emb_dim, nt, 128), jnp.float32),
        mesh=mesh,
        compiler_params=pltpu.CompilerParams(needs_layout_passes=False),
        scratch_types=[
            pltpu.VMEM((vocab,), jnp.float32),
            pltpu.VMEM((2, ichunk), jnp.int32),
            pltpu.VMEM((2, ichunk), jnp.int32),
            pltpu.VMEM((2, ichunk // 128, 128), jnp.float32),
            pltpu.VMEM((2, ichunk // 128, 128), jnp.float32),
            pltpu.SemaphoreType.DMA,
            pltpu.SemaphoreType.DMA,
            pltpu.SemaphoreType.DMA,
            pltpu.SemaphoreType.DMA,
            pltpu.SemaphoreType.DMA,
            pltpu.SemaphoreType.DMA,
            pltpu.SemaphoreType.DMA,
            pltpu.SemaphoreType.DMA,
            pltpu.SemaphoreType.DMA,
        ],
    )
    def sc_gather(tT_hbm, ia_hbm, ib_hbm, rows_hbm,
                  row_v, ia_v, ib_v, ra_v, rb_v,
                  sem_row, sem_ia0, sem_ia1, sem_ib0, sem_ib1,
                  sem_ra0, sem_ra1, sem_rb0, sem_rb1):
        c = lax.axis_index("c")
        s = lax.axis_index("s")
        d = c * half + s
        sem_ia = [sem_ia0, sem_ia1]
        sem_ib = [sem_ib0, sem_ib1]
        sem_ra = [sem_ra0, sem_ra1]
        sem_rb = [sem_rb0, sem_rb1]

        h_row = pltpu.async_copy(tT_hbm.at[d], row_v, sem_row)

        nch = batch // ichunk
        tpc = ichunk // 128

        def start_in(k):
            sl = k % 2
            return (
                pltpu.async_copy(ia_hbm.at[pl.ds(k * ichunk, ichunk)],
                                 ia_v.at[sl], sem_ia[sl]),
                pltpu.async_copy(ib_hbm.at[pl.ds(k * ichunk, ichunk)],
                                 ib_v.at[sl], sem_ib[sl]),
            )

        h_in = start_in(0)
        h_row.wait()
        h_out = [None] * nch
        for k in range(nch):
            sl = k % 2
            h_in[0].wait()
            h_in[1].wait()
            if k + 1 < nch:
                h_in = start_in(k + 1)
            if k >= 2:
                h_out[k - 2][0].wait()
                h_out[k - 2][1].wait()

            @plsc.parallel_loop(0, ichunk // 16, unroll=16)
            def gather_one(j):
                off = j * 16
                t, l = off // 128, off % 128
                ra_v[sl, t, pl.ds(l, 16)] = plsc.load_gather(
                    row_v, [ia_v[sl, pl.ds(off, 16)]])
                rb_v[sl, t, pl.ds(l, 16)] = plsc.load_gather(
                    row_v, [ib_v[sl, pl.ds(off, 16)]])

            h_out[k] = (
                pltpu.async_copy(ra_v.at[sl],
                                 rows_hbm.at[d, pl.ds(k * tpc, tpc)],
                                 sem_ra[sl]),
                pltpu.async_copy(rb_v.at[sl],
                                 rows_hbm.at[emb_dim + d, pl.ds(k * tpc, tpc)],
                                 sem_rb[sl]),
            )
        for k in (nch - 2, nch - 1):
            h_out[k][0].wait()
            h_out[k][1].wait()

    return sc_gather


def _make_sc_xpose(batch, emb_dim):
    info = plsc.get_sparse_core_info()
    nw = info.num_cores * info.num_subcores
    nt = batch // 128
    per_w = batch // nw
    tpw = per_w // 128
    two_d = 2 * emb_dim
    mesh = plsc.VectorSubcoreMesh(core_axis_name="c", subcore_axis_name="s")

    @functools.partial(
        pl.kernel,
        out_type=jax.ShapeDtypeStruct((batch, 128), jnp.float32),
        mesh=mesh,
        compiler_params=pltpu.CompilerParams(use_tc_tiling_on_sc=False,
                                             needs_layout_passes=False),
        scratch_types=[
            pltpu.VMEM((two_d, tpw, 128), jnp.float32),
            pltpu.VMEM((per_w, 129), jnp.float32),
        ],
    )
    def sc_xpose(rows_hbm, comb_hbm, xch_v, out_t):
        wid = lax.axis_index("s") * info.num_cores + lax.axis_index("c")
        b0 = wid * per_w
        pltpu.sync_copy(rows_hbm.at[:, pl.ds(wid * tpw, tpw)], xch_v)
        lane16 = lax.broadcasted_iota(jnp.int32, (16,), 0)

        @plsc.parallel_loop(0, two_d * per_w // 16, unroll=8)
        def xpose_one(t):
            r = t % two_d
            bb = (t // two_d) * 16
            vals = xch_v[r, bb // 128, pl.ds(bb % 128, 16)]
            plsc.store_scatter(
                out_t, [bb + lane16, jnp.full((16,), r, jnp.int32)], vals)
        pltpu.sync_copy(out_t.at[:, pl.ds(0, 128)], comb_hbm.at[pl.ds(b0, per_w)])

    return sc_xpose


_sc_gather_t = _make_sc_gather_t(_BATCH, _EMB, 100000)
_sc_xpose = _make_sc_xpose(_BATCH, _EMB)


def _dot_t(a, w):
    return lax.dot_general(a, w, (((1,), (1,)), ((), ())),
                           preferred_element_type=jnp.float32)


def _mlp_body(comb, ft, w1, b1, w2, b2, w3, b3, w4, out):
    x = jnp.concatenate([comb[:, 0:64], ft[...]], axis=1)
    h = jnp.maximum(_dot_t(x, w1[...]) + b1[...], 0.0)
    h = jnp.maximum(_dot_t(h, w2[...]) + b2[...], 0.0)
    h = jnp.maximum(_dot_t(h, w3[...]) + b3[...], 0.0)
    h = jnp.concatenate([h, jnp.ones((h.shape[0], 1), jnp.float32)], axis=1)
    out[...] = jax.nn.sigmoid(_dot_t(w4[...], h))


def _mlp(comb, ft, W1, b1, W2, b2, W3, b3, W4e, blk=2048):
    batch = comb.shape[0]
    grid = (batch // blk,)
    full = lambda a: pl.BlockSpec(a.shape, lambda i: (0,) * a.ndim)
    row = lambda a: pl.BlockSpec((blk, a.shape[1]), lambda i: (i, 0))
    return pl.pallas_call(
        _mlp_body,
        grid=grid,
        in_specs=[
            row(comb), row(ft),
            full(W1), full(b1), full(W2), full(b2),
            full(W3), full(b3), full(W4e),
        ],
        out_specs=pl.BlockSpec((1, blk), lambda i: (0, i)),
        out_shape=jax.ShapeDtypeStruct((1, batch), jnp.float32),
    )(comb, ft, W1, b1, W2, b2, W3, b3, W4e)


def kernel(airport_a, airport_b, features, table,
           W1, b1, W2, b2, W3, b3, W4, b4):
    ia = airport_a.astype(jnp.int32)
    ib = airport_b.astype(jnp.int32)
    rows = _sc_gather_t(table.T, ia, ib)
    comb = _sc_xpose(rows)
    w4e = jnp.concatenate([W4, b4.reshape(1, 1)], axis=1)
    out = _mlp(comb, features,
               W1, b1.reshape(1, -1), W2, b2.reshape(1, -1),
               W3, b3.reshape(1, -1), w4e)
    return out.reshape(-1)
